# baseline (speedup 1.0000x reference)
import functools

import jax
import jax.numpy as jnp
from jax import lax
from jax.experimental import pallas as pl
from jax.experimental.pallas import tpu as pltpu
from jax.experimental.pallas import tpu_sc as plsc

_CH = 128
_NTILES = 32
_NSUB = 16
_NG = 4


def _enc_body(x_ref, we_ref, be_ref, ws_ref, x0_ref, p_ref):
    x0 = jnp.dot(x_ref[...], we_ref[...],
                 preferred_element_type=jnp.float32) + be_ref[...]
    x0_ref[...] = x0
    p_ref[...] = jnp.dot(x0, ws_ref[...], preferred_element_type=jnp.float32)


def _mid_body(h_ref, x0_ref, o_ref, ws_ref, om_ref, beta_ref, h2_ref, p2_ref):
    h = h_ref[...]
    pre = o_ref[...] - h * om_ref[...] - beta_ref[0, 0] * x0_ref[...]
    h2 = h + jnp.maximum(pre, 0.0)
    h2_ref[...] = h2
    p2_ref[...] = jnp.dot(h2, ws_ref[...], preferred_element_type=jnp.float32)


def _dec_body(h_ref, x0_ref, o_ref, om_ref, beta_ref, wd_ref, bd_ref, y_ref):
    h = h_ref[...]
    pre = o_ref[...] - h * om_ref[...] - beta_ref[0, 0] * x0_ref[...]
    h3 = h + jnp.maximum(pre, 0.0)
    y_ref[...] = jnp.dot(h3, wd_ref[...],
                         preferred_element_type=jnp.float32) + bd_ref[...]


def _row_spec(blk, d):
    return pl.BlockSpec((blk, d), lambda i: (i, 0))


def _full_spec(shape):
    ndim = len(shape)
    return pl.BlockSpec(shape, lambda i: (0,) * ndim)


def _sc_aggregate(p_tab, ei_t, zeros_rpt):
    np2, dh = p_tab.shape
    np_ = np2 // 2
    _, _, k, ch = ei_t.shape
    rpt = np_ // _NSUB

    mesh = plsc.VectorSubcoreMesh(core_axis_name="c", subcore_axis_name="s")

    @functools.partial(
        pl.kernel,
        mesh=mesh,
        out_type=jax.ShapeDtypeStruct((np_, 2, dh), jnp.float32),
        compiler_params=pltpu.CompilerParams(use_tc_tiling_on_sc=False),
        scratch_types=[
            pltpu.VMEM((2, k, ch), jnp.int32),
            *[pltpu.VMEM((ch, dh), jnp.float32) for _ in range(_NG)],
            pltpu.VMEM_SHARED((np_, dh), jnp.float32),
            pltpu.SemaphoreType.DMA,
            pltpu.SemaphoreType.DMA,
        ],
    )
    def agg_kernel(p_hbm, ei_hbm, zero_hbm, out_hbm, ei_v, *rest):
        bufs = rest[:_NG]
        agg_sh = rest[_NG]
        gsem, ssem = rest[_NG + 1], rest[_NG + 2]
        c = lax.axis_index("c")
        s = lax.axis_index("s")
        w = c * _NSUB + s
        row0 = s * rpt

        pltpu.sync_copy(zero_hbm, agg_sh.at[pl.ds(row0, rpt)])
        pltpu.sync_copy(ei_hbm.at[w], ei_v)
        plsc.subcore_barrier()

        def body(t, carry):
            base = t * _NG
            gts = [
                pltpu.async_copy(p_hbm.at[ei_v.at[0, base + b]], bufs[b],
                                 gsem)
                for b in range(_NG)
            ]
            for gt in gts:
                gt.wait()
            sts = [
                pltpu.async_copy(bufs[b], agg_sh.at[ei_v.at[1, base + b]],
                                 ssem, add=True)
                for b in range(_NG)
            ]
            for st in sts:
                st.wait()
            return carry

        lax.fori_loop(0, k // _NG, body, 0)
        plsc.subcore_barrier()

        pltpu.sync_copy(agg_sh.at[pl.ds(row0, rpt)],
                        out_hbm.at[pl.ds(row0, rpt), c])

    return agg_kernel(p_tab, ei_t, zeros_rpt)


def kernel(x, edge_index, W_enc, b_enc, W_dec, b_dec, W_pair, omega, beta):
    n, d = x.shape
    e = edge_index.shape[1]
    num_layers = 3

    rpt = -(-(n + 1) // (_NSUB * 8)) * 8
    np_ = _NSUB * rpt
    blk = np_ // 16
    dh = d // 2
    grp2 = 2 * _NG
    k = -(-e // (_NSUB * _CH * grp2)) * grp2
    k_idx = k

    x_pad = jnp.zeros((np_, d), jnp.float32).at[:n].set(x)
    ws = 0.5 * (W_pair + W_pair.T)
    src = edge_index[0].astype(jnp.int32)
    dst = edge_index[1].astype(jnp.int32)
    npad = _NSUB * k * _CH - e
    pad_src = (jnp.arange(npad, dtype=jnp.int32) * 97) % np_
    pad_dst = n + jnp.arange(npad, dtype=jnp.int32) % (np_ - n)
    srcb = jnp.concatenate([src, pad_src]).reshape(1, _NSUB, 1, k, _CH)
    dstb = jnp.concatenate([dst, pad_dst]).reshape(1, _NSUB, 1, k, _CH)
    src2 = jnp.concatenate([2 * srcb, 2 * srcb + 1])
    dst2 = jnp.concatenate([dstb, dstb])
    ei_t = jnp.concatenate([src2, dst2], axis=2).reshape(_NTILES, 2, k, _CH)
    zeros_rpt = jnp.zeros((rpt, dh), jnp.float32)
    b_enc2 = b_enc.reshape(1, d)
    b_dec2 = b_dec.reshape(1, d)
    om2 = omega.reshape(1, d)
    beta2 = jnp.reshape(beta, (1, 1)).astype(jnp.float32)

    grid = (np_ // blk,)
    row = _row_spec(blk, d)
    o_spec = row

    x0, p = pl.pallas_call(
        _enc_body,
        grid=grid,
        in_specs=[row, _full_spec((d, d)), _full_spec((1, d)),
                  _full_spec((d, d))],
        out_specs=[row, row],
        out_shape=[jax.ShapeDtypeStruct((np_, d), jnp.float32)] * 2,
    )(x_pad, W_enc, b_enc2, ws)

    h = x0
    y = None
    for layer in range(num_layers):
        parts = _sc_aggregate(p.reshape(2 * np_, dh), ei_t, zeros_rpt)
        o2 = parts.reshape(np_, d)
        if layer < num_layers - 1:
            h, p = pl.pallas_call(
                _mid_body,
                grid=grid,
                in_specs=[row, row, o_spec, _full_spec((d, d)),
                          _full_spec((1, d)), _full_spec((1, 1))],
                out_specs=[row, row],
                out_shape=[jax.ShapeDtypeStruct((np_, d), jnp.float32)] * 2,
            )(h, x0, o2, ws, om2, beta2)
        else:
            y = pl.pallas_call(
                _dec_body,
                grid=grid,
                in_specs=[row, row, o_spec, _full_spec((1, d)),
                          _full_spec((1, 1)), _full_spec((d, d)),
                          _full_spec((1, d))],
                out_specs=row,
                out_shape=jax.ShapeDtypeStruct((np_, d), jnp.float32),
            )(h, x0, o2, om2, beta2, W_dec, b_dec2)

    return y[:n]


# speedup vs baseline: 1.0309x; 1.0309x over previous
import functools

import jax
import jax.numpy as jnp
from jax import lax
from jax.experimental import pallas as pl
from jax.experimental.pallas import tpu as pltpu
from jax.experimental.pallas import tpu_sc as plsc

_CH = 128
_NTILES = 32
_NSUB = 16
_NG = 4


def _enc_body(x_ref, we_ref, be_ref, ws_ref, x0_ref, p_ref):
    x0 = jnp.dot(x_ref[...], we_ref[...],
                 preferred_element_type=jnp.float32) + be_ref[...]
    x0_ref[...] = x0
    p_ref[...] = jnp.dot(x0, ws_ref[...], preferred_element_type=jnp.float32)


def _mid_body(h_ref, x0_ref, o_ref, ws_ref, om_ref, beta_ref, h2_ref, p2_ref):
    h = h_ref[...]
    pre = o_ref[...] - h * om_ref[...] - beta_ref[0, 0] * x0_ref[...]
    h2 = h + jnp.maximum(pre, 0.0)
    h2_ref[...] = h2
    p2_ref[...] = jnp.dot(h2, ws_ref[...], preferred_element_type=jnp.float32)


def _dec_body(h_ref, x0_ref, o_ref, om_ref, beta_ref, wd_ref, bd_ref, y_ref):
    h = h_ref[...]
    pre = o_ref[...] - h * om_ref[...] - beta_ref[0, 0] * x0_ref[...]
    h3 = h + jnp.maximum(pre, 0.0)
    y_ref[...] = jnp.dot(h3, wd_ref[...],
                         preferred_element_type=jnp.float32) + bd_ref[...]


def _row_spec(blk, d):
    return pl.BlockSpec((blk, d), lambda i: (i, 0))


def _full_spec(shape):
    ndim = len(shape)
    return pl.BlockSpec(shape, lambda i: (0,) * ndim)


def _sc_aggregate(p_tab, ei_t, zeros_rpt):
    np2, dh = p_tab.shape
    np_ = np2 // 2
    _, _, k, ch = ei_t.shape
    rpt = np_ // _NSUB

    mesh = plsc.VectorSubcoreMesh(core_axis_name="c", subcore_axis_name="s")

    @functools.partial(
        pl.kernel,
        mesh=mesh,
        out_type=jax.ShapeDtypeStruct((np_, 2, dh), jnp.float32),
        compiler_params=pltpu.CompilerParams(use_tc_tiling_on_sc=False),
        scratch_types=[
            pltpu.VMEM((2, k, ch), jnp.int32),
            *[pltpu.VMEM((ch, dh), jnp.float32) for _ in range(_NG)],
            pltpu.VMEM_SHARED((np_, dh), jnp.float32),
            pltpu.SemaphoreType.DMA,
            pltpu.SemaphoreType.DMA,
        ],
    )
    def agg_kernel(p_hbm, ei_hbm, zero_hbm, out_hbm, ei_v, *rest):
        bufs = rest[:_NG]
        agg_sh = rest[_NG]
        gsem, ssem = rest[_NG + 1], rest[_NG + 2]
        c = lax.axis_index("c")
        s = lax.axis_index("s")
        w = c * _NSUB + s
        row0 = s * rpt

        pltpu.sync_copy(zero_hbm, bufs[0])
        for r in range(rpt // ch):
            pltpu.sync_copy(bufs[0], agg_sh.at[pl.ds(row0 + r * ch, ch)])
        pltpu.sync_copy(ei_hbm.at[w], ei_v)
        plsc.subcore_barrier()

        def body(t, carry):
            base = t * _NG
            gts = [
                pltpu.async_copy(p_hbm.at[ei_v.at[0, base + b]], bufs[b],
                                 gsem)
                for b in range(_NG)
            ]
            for gt in gts:
                gt.wait()
            sts = [
                pltpu.async_copy(bufs[b], agg_sh.at[ei_v.at[1, base + b]],
                                 ssem, add=True)
                for b in range(_NG)
            ]
            for st in sts:
                st.wait()
            return carry

        lax.fori_loop(0, k // _NG, body, 0)
        plsc.subcore_barrier()

        pltpu.sync_copy(agg_sh.at[pl.ds(row0, rpt)],
                        out_hbm.at[pl.ds(row0, rpt), c])

    return agg_kernel(p_tab, ei_t, zeros_rpt)


def kernel(x, edge_index, W_enc, b_enc, W_dec, b_dec, W_pair, omega, beta):
    n, d = x.shape
    e = edge_index.shape[1]
    num_layers = 3

    np_ = ((n + 2047) // 2048) * 2048
    rpt = np_ // _NSUB
    blk = np_ // 10
    dh = d // 2
    grp2 = 2 * _NG
    k = -(-e // (_NSUB * _CH * grp2)) * grp2
    k_idx = k

    x_pad = jnp.zeros((np_, d), jnp.float32).at[:n].set(x)
    ws = 0.5 * (W_pair + W_pair.T)
    src = edge_index[0].astype(jnp.int32)
    dst = edge_index[1].astype(jnp.int32)
    npad = _NSUB * k * _CH - e
    pad_src = (jnp.arange(npad, dtype=jnp.int32) * 97) % np_
    pad_dst = n + jnp.arange(npad, dtype=jnp.int32) % (np_ - n)
    srcb = jnp.concatenate([src, pad_src]).reshape(1, _NSUB, 1, k, _CH)
    dstb = jnp.concatenate([dst, pad_dst]).reshape(1, _NSUB, 1, k, _CH)
    src2 = jnp.concatenate([2 * srcb, 2 * srcb + 1])
    dst2 = jnp.concatenate([dstb, dstb])
    ei_t = jnp.concatenate([src2, dst2], axis=2).reshape(_NTILES, 2, k, _CH)
    zeros_chunk = jnp.zeros((_CH, dh), jnp.float32)
    b_enc2 = b_enc.reshape(1, d)
    b_dec2 = b_dec.reshape(1, d)
    om2 = omega.reshape(1, d)
    beta2 = jnp.reshape(beta, (1, 1)).astype(jnp.float32)

    grid = (np_ // blk,)
    row = _row_spec(blk, d)
    o_spec = row

    x0, p = pl.pallas_call(
        _enc_body,
        grid=grid,
        in_specs=[row, _full_spec((d, d)), _full_spec((1, d)),
                  _full_spec((d, d))],
        out_specs=[row, row],
        out_shape=[jax.ShapeDtypeStruct((np_, d), jnp.float32)] * 2,
    )(x_pad, W_enc, b_enc2, ws)

    h = x0
    y = None
    for layer in range(num_layers):
        parts = _sc_aggregate(p.reshape(2 * np_, dh), ei_t, zeros_chunk)
        o2 = parts.reshape(np_, d)
        if layer < num_layers - 1:
            h, p = pl.pallas_call(
                _mid_body,
                grid=grid,
                in_specs=[row, row, o_spec, _full_spec((d, d)),
                          _full_spec((1, d)), _full_spec((1, 1))],
                out_specs=[row, row],
                out_shape=[jax.ShapeDtypeStruct((np_, d), jnp.float32)] * 2,
            )(h, x0, o2, ws, om2, beta2)
        else:
            y = pl.pallas_call(
                _dec_body,
                grid=grid,
                in_specs=[row, row, o_spec, _full_spec((1, d)),
                          _full_spec((1, 1)), _full_spec((d, d)),
                          _full_spec((1, d))],
                out_specs=row,
                out_shape=jax.ShapeDtypeStruct((np_, d), jnp.float32),
            )(h, x0, o2, om2, beta2, W_dec, b_dec2)

    return y[:n]
